# baseline (speedup 1.0000x reference)
import functools

import jax
import jax.numpy as jnp
from jax import lax
from jax.experimental import pallas as pl
from jax.experimental.pallas import tpu as pltpu
from jax.experimental.pallas import tpu_sc as plsc

_B, _L, _D = 4096, 200, 64
_HALF = _L // 2
_NC, _NS = 2, 16
_NW = _NC * _NS
_BPW = _B // _NW
_CH = _D // 16


def _bag_body(x_hbm, table_hbm, out_hbm, idx_v, rows_v, acc_v, sem0, sem1):
    wid = lax.axis_index("s") * _NC + lax.axis_index("c")
    base = wid * _BPW
    pltpu.sync_copy(x_hbm.at[pl.ds(base, _BPW)], idx_v)

    def issue(bag, buf, sem):
        pltpu.async_copy(table_hbm.at[idx_v.at[bag, 0]],
                         rows_v.at[buf, pl.ds(0, _HALF)], sem)
        pltpu.async_copy(table_hbm.at[idx_v.at[bag, 1]],
                         rows_v.at[buf, pl.ds(_HALF, _HALF)], sem)

    def wait(buf, sem):
        pltpu.make_async_copy(table_hbm.at[pl.ds(0, _L)],
                              rows_v.at[buf], sem).wait()

    def accumulate(buf, row):
        def rbody(l, accs):
            return tuple(accs[c] + rows_v[buf, l, pl.ds(16 * c, 16)]
                         for c in range(_CH))
        accs = lax.fori_loop(
            0, _L, rbody,
            tuple(jnp.zeros((16,), jnp.float32) for _ in range(_CH)))
        for c in range(_CH):
            acc_v[row, pl.ds(16 * c, 16)] = accs[c]

    issue(0, 0, sem0)

    def pair(g, carry):
        bag0 = 2 * g
        issue(bag0 + 1, 1, sem1)
        wait(0, sem0)
        accumulate(0, bag0)

        @pl.when(bag0 + 2 < _BPW)
        def _():
            issue(bag0 + 2, 0, sem0)

        wait(1, sem1)
        accumulate(1, bag0 + 1)
        return carry

    lax.fori_loop(0, _BPW // 2, pair, 0)
    pltpu.sync_copy(acc_v, out_hbm.at[pl.ds(base, _BPW)])


_bag_sum = functools.partial(
    pl.kernel,
    out_type=jax.ShapeDtypeStruct((_B, _D), jnp.float32),
    mesh=plsc.VectorSubcoreMesh(core_axis_name="c", subcore_axis_name="s"),
    scratch_types=[
        pltpu.VMEM((_BPW, 2, _HALF), jnp.int32),
        pltpu.VMEM((2, _L, _D), jnp.float32),
        pltpu.VMEM((_BPW, _D), jnp.float32),
        pltpu.SemaphoreType.DMA,
        pltpu.SemaphoreType.DMA,
    ],
    compiler_params=pltpu.CompilerParams(use_tc_tiling_on_sc=False),
)(_bag_body)


def _mlp_body(bag_ref, w1_ref, b1_ref, w2_ref, b2_ref, out_ref):
    emb = bag_ref[...] * (1.0 / _L)
    h = jnp.dot(emb, w1_ref[...], preferred_element_type=jnp.float32)
    h = jnp.maximum(h + b1_ref[...], 0.0)
    logits = jnp.dot(h, w2_ref[...], preferred_element_type=jnp.float32)
    logits = logits + b2_ref[...]
    m = jnp.max(logits, axis=1, keepdims=True)
    e = jnp.exp(logits - m)
    out_ref[...] = e / jnp.sum(e, axis=1, keepdims=True)


def kernel(x, table, W1, b1, W2, b2):
    nc = W2.shape[1]
    xi = x.astype(jnp.int32).reshape(_B, 2, _HALF)
    bag = _bag_sum(xi, table)
    return pl.pallas_call(
        _mlp_body,
        out_shape=jax.ShapeDtypeStruct((_B, nc), jnp.float32),
    )(bag, W1, b1.reshape(1, _D), W2, b2.reshape(1, nc))
